# baseline (speedup 1.0000x reference)
import functools

import jax
import jax.numpy as jnp
from jax import lax
from jax.experimental import pallas as pl
from jax.experimental.pallas import tpu as pltpu
from jax.experimental.pallas import tpu_sc as plsc

N_NODES = 10000
N_EDGES = 160000
D_IN = 256
D_OUT = 256
DH = D_IN // 2
NS = 16
EPS = N_EDGES // NS
CH = 104
NCHUNK = 97
EPSP = NCHUNK * CH
NPAD = EPSP - EPS
TRASH = 8
ROWS0 = 624
ROWS_LAST = N_NODES - 15 * ROWS0

_mesh = plsc.VectorSubcoreMesh(core_axis_name="c", subcore_axis_name="s")


@functools.partial(
    pl.kernel,
    mesh=_mesh,
    out_type=(
        jax.ShapeDtypeStruct((N_NODES, DH), jnp.float32),
        jax.ShapeDtypeStruct((N_NODES, DH), jnp.float32),
    ),
    scratch_types=[
        pltpu.VMEM((EPSP,), jnp.int32),
        pltpu.VMEM((NCHUNK, CH), jnp.int32),
        pltpu.VMEM((CH, DH), jnp.float32),
        pltpu.VMEM((CH, DH), jnp.float32),
        pltpu.VMEM_SHARED((N_NODES + TRASH, DH), jnp.float32),
        pltpu.SemaphoreType.DMA,
        pltpu.SemaphoreType.DMA,
    ],
)
def _scatter_sum(x2, src_l, src_r, dst_p, zeros, h_l, h_r, src_v, dst_v,
                 rows0_v, rows1_v, h_sh, sem0, sem1):
    c = lax.axis_index("c")
    s = lax.axis_index("s")
    base = pl.multiple_of(s * ROWS0, 8)

    @pl.when(s < 15)
    def _():
        pltpu.sync_copy(zeros.at[pl.ds(0, ROWS0)],
                        h_sh.at[pl.ds(base, ROWS0)])

    @pl.when(s == 15)
    def _():
        pltpu.sync_copy(zeros, h_sh.at[pl.ds(15 * ROWS0, ROWS_LAST)])

    @pl.when(c == 0)
    def _():
        pltpu.sync_copy(src_l.at[s], src_v)

    @pl.when(c == 1)
    def _():
        pltpu.sync_copy(src_r.at[s], src_v)

    pltpu.sync_copy(dst_p.at[s], dst_v)
    plsc.subcore_barrier()

    def sidx(k):
        return src_v.at[pl.ds(pl.multiple_of(k * CH, 8), CH)]

    pltpu.async_copy(x2.at[sidx(0)], rows0_v, sem0)
    pltpu.async_copy(x2.at[sidx(1)], rows1_v, sem1)

    def body(j, carry):
        a = 2 * j
        pltpu.make_async_copy(x2.at[pl.ds(0, CH)], rows0_v, sem0).wait()
        pltpu.sync_copy(rows0_v, h_sh.at[dst_v.at[a]], add=True)
        pltpu.async_copy(x2.at[sidx(a + 2)], rows0_v, sem0)
        pltpu.make_async_copy(x2.at[pl.ds(0, CH)], rows1_v, sem1).wait()
        pltpu.sync_copy(rows1_v, h_sh.at[dst_v.at[a + 1]], add=True)

        @pl.when(j < (NCHUNK - 1) // 2 - 1)
        def _():
            pltpu.async_copy(x2.at[sidx(a + 3)], rows1_v, sem1)
        return carry

    lax.fori_loop(0, (NCHUNK - 1) // 2, body, 0)
    pltpu.make_async_copy(x2.at[pl.ds(0, CH)], rows0_v, sem0).wait()
    pltpu.sync_copy(rows0_v, h_sh.at[dst_v.at[NCHUNK - 1]], add=True)

    plsc.subcore_barrier()

    h_out = [h_l, h_r]
    for ci, h_hbm in enumerate(h_out):
        @pl.when((c == ci) & (s < 15))
        def _(h_hbm=h_hbm):
            row = pl.ds(base, ROWS0)
            pltpu.sync_copy(h_sh.at[row], h_hbm.at[row])

        @pl.when((c == ci) & (s == 15))
        def _(h_hbm=h_hbm):
            row = pl.ds(15 * ROWS0, ROWS_LAST)
            pltpu.sync_copy(h_sh.at[row], h_hbm.at[row])


BR = 1000


def _mm_body(hl_ref, hr_ref, wl_ref, wr_ref, b_ref, o_ref):
    acc = lax.dot_general(hl_ref[...], wl_ref[...],
                          (((1,), (1,)), ((), ())),
                          preferred_element_type=jnp.float32)
    acc = acc + lax.dot_general(hr_ref[...], wr_ref[...],
                                (((1,), (1,)), ((), ())),
                                preferred_element_type=jnp.float32)
    o_ref[...] = jnp.maximum(acc + b_ref[...], 0.0)


_matmul = pl.pallas_call(
    _mm_body,
    grid=(N_NODES // BR,),
    in_specs=[
        pl.BlockSpec((BR, DH), lambda i: (i, 0)),
        pl.BlockSpec((BR, DH), lambda i: (i, 0)),
        pl.BlockSpec((D_OUT, DH), lambda i: (0, 0)),
        pl.BlockSpec((D_OUT, DH), lambda i: (0, 0)),
        pl.BlockSpec((1, D_OUT), lambda i: (0, 0)),
    ],
    out_specs=pl.BlockSpec((BR, D_OUT), lambda i: (i, 0)),
    out_shape=jax.ShapeDtypeStruct((N_NODES, D_OUT), jnp.float32),
)


def kernel(x, edge_index, W, b):
    x2 = x.reshape(2 * N_NODES, DH)
    pad_src = jnp.broadcast_to(
        jnp.arange(NPAD, dtype=jnp.int32) % 16, (NS, NPAD))
    pad_dst = jnp.broadcast_to(
        N_NODES + (jnp.arange(NPAD, dtype=jnp.int32) % TRASH), (NS, NPAD))
    src_b = jnp.concatenate(
        [edge_index[0].reshape(NS, EPS), pad_src], axis=1)
    src_l = src_b * 2
    src_r = src_b * 2 + 1
    dst_p = jnp.concatenate(
        [edge_index[1].reshape(NS, EPS), pad_dst], axis=1).reshape(
            NS, NCHUNK, CH)
    zeros = jnp.zeros((ROWS_LAST, DH), jnp.float32)
    h_l, h_r = _scatter_sum(x2, src_l, src_r, dst_p, zeros)
    w_l = W[:, :DH]
    w_r = W[:, DH:]
    return _matmul(h_l, h_r, w_l, w_r, b.reshape(1, D_OUT))


# speedup vs baseline: 1.0773x; 1.0773x over previous
import functools

import jax
import jax.numpy as jnp
from jax import lax
from jax.experimental import pallas as pl
from jax.experimental.pallas import tpu as pltpu
from jax.experimental.pallas import tpu_sc as plsc

N_NODES = 10000
N_EDGES = 160000
D_IN = 256
D_OUT = 256
DH = D_IN // 2
NS = 16
EPS = N_EDGES // NS
CH = 112
NCHUNK = 90
EPSP = NCHUNK * CH
NPAD = EPSP - EPS
X_PAD = 16
ROWS0 = 624
ROWS_LAST = N_NODES - 15 * ROWS0

_mesh = plsc.VectorSubcoreMesh(core_axis_name="c", subcore_axis_name="s")


@functools.partial(
    pl.kernel,
    mesh=_mesh,
    out_type=(
        jax.ShapeDtypeStruct((N_NODES, DH), jnp.float32),
        jax.ShapeDtypeStruct((N_NODES, DH), jnp.float32),
    ),
    scratch_types=[
        pltpu.VMEM((EPSP,), jnp.int32),
        pltpu.VMEM((NCHUNK, CH), jnp.int32),
        pltpu.VMEM((CH, DH), jnp.float32),
        pltpu.VMEM((CH, DH), jnp.float32),
        pltpu.VMEM_SHARED((N_NODES, DH), jnp.float32),
        pltpu.SemaphoreType.DMA,
        pltpu.SemaphoreType.DMA,
    ],
)
def _scatter_sum(x_l, x_r, src_p, dst_p, h_l, h_r, src_v, dst_v,
                 rows0_v, rows1_v, h_sh, sem0, sem1):
    c = lax.axis_index("c")
    s = lax.axis_index("s")
    base = pl.multiple_of(s * ROWS0, 8)

    def zbody(r, carry):
        z = jnp.zeros((16,), jnp.float32)
        for k in range(DH // 16):
            rows0_v[r, pl.ds(16 * k, 16)] = z
        return carry

    lax.fori_loop(0, CH, zbody, 0)
    for t in range(5):
        pltpu.sync_copy(rows0_v, h_sh.at[pl.ds(base + t * CH, CH)])

    @pl.when(s < 15)
    def _():
        pltpu.sync_copy(rows0_v.at[pl.ds(0, ROWS0 - 5 * CH)],
                        h_sh.at[pl.ds(base + 5 * CH, ROWS0 - 5 * CH)])

    @pl.when(s == 15)
    def _():
        pltpu.sync_copy(rows0_v.at[pl.ds(0, ROWS_LAST - 5 * CH)],
                        h_sh.at[pl.ds(15 * ROWS0 + 5 * CH,
                                      ROWS_LAST - 5 * CH)])

    pltpu.sync_copy(src_p.at[s], src_v)
    pltpu.sync_copy(dst_p.at[s], dst_v)
    plsc.subcore_barrier()

    def _run(x_hbm):
        def sidx(k):
            return src_v.at[pl.ds(pl.multiple_of(k * CH, 8), CH)]

        pltpu.async_copy(x_hbm.at[sidx(0)], rows0_v, sem0)
        pltpu.async_copy(x_hbm.at[sidx(1)], rows1_v, sem1)

        def body(j, carry):
            a = 2 * j
            pltpu.make_async_copy(x_hbm.at[pl.ds(0, CH)], rows0_v, sem0).wait()
            pltpu.sync_copy(rows0_v, h_sh.at[dst_v.at[a]], add=True)

            @pl.when(j < NCHUNK // 2 - 1)
            def _():
                pltpu.async_copy(x_hbm.at[sidx(a + 2)], rows0_v, sem0)

            pltpu.make_async_copy(x_hbm.at[pl.ds(0, CH)], rows1_v, sem1).wait()
            pltpu.sync_copy(rows1_v, h_sh.at[dst_v.at[a + 1]], add=True)

            @pl.when(j < NCHUNK // 2 - 1)
            def _():
                pltpu.async_copy(x_hbm.at[sidx(a + 3)], rows1_v, sem1)
            return carry

        lax.fori_loop(0, NCHUNK // 2, body, 0)

    @pl.when(c == 0)
    def _():
        _run(x_l)

    @pl.when(c == 1)
    def _():
        _run(x_r)

    plsc.subcore_barrier()

    h_out = [h_l, h_r]
    for ci, h_hbm in enumerate(h_out):
        @pl.when((c == ci) & (s < 15))
        def _(h_hbm=h_hbm):
            row = pl.ds(base, ROWS0)
            pltpu.sync_copy(h_sh.at[row], h_hbm.at[row])

        @pl.when((c == ci) & (s == 15))
        def _(h_hbm=h_hbm):
            row = pl.ds(15 * ROWS0, ROWS_LAST)
            pltpu.sync_copy(h_sh.at[row], h_hbm.at[row])


BR = 2000


def _mm_body(hl_ref, hr_ref, wl_ref, wr_ref, b_ref, o_ref):
    acc = lax.dot_general(hl_ref[...], wl_ref[...],
                          (((1,), (1,)), ((), ())),
                          preferred_element_type=jnp.float32)
    acc = acc + lax.dot_general(hr_ref[...], wr_ref[...],
                                (((1,), (1,)), ((), ())),
                                preferred_element_type=jnp.float32)
    o_ref[...] = jnp.maximum(acc + b_ref[...], 0.0)


_matmul = pl.pallas_call(
    _mm_body,
    grid=(N_NODES // BR,),
    in_specs=[
        pl.BlockSpec((BR, DH), lambda i: (i, 0)),
        pl.BlockSpec((BR, DH), lambda i: (i, 0)),
        pl.BlockSpec((D_OUT, DH), lambda i: (0, 0)),
        pl.BlockSpec((D_OUT, DH), lambda i: (0, 0)),
        pl.BlockSpec((1, D_OUT), lambda i: (0, 0)),
    ],
    out_specs=pl.BlockSpec((BR, D_OUT), lambda i: (i, 0)),
    out_shape=jax.ShapeDtypeStruct((N_NODES, D_OUT), jnp.float32),
)


def kernel(x, edge_index, W, b):
    xz = jnp.zeros((X_PAD, DH), jnp.float32)
    x_l = jnp.concatenate([x[:, :DH], xz], axis=0)
    x_r = jnp.concatenate([x[:, DH:], xz], axis=0)
    pad_src = jnp.broadcast_to(
        N_NODES + (jnp.arange(NPAD, dtype=jnp.int32) % X_PAD), (NS, NPAD))
    pad_dst = jnp.zeros((NS, NPAD), jnp.int32)
    src_p = jnp.concatenate(
        [edge_index[0].reshape(NS, EPS), pad_src], axis=1)
    dst_p = jnp.concatenate(
        [edge_index[1].reshape(NS, EPS), pad_dst], axis=1).reshape(
            NS, NCHUNK, CH)
    h_l, h_r = _scatter_sum(x_l, x_r, src_p, dst_p)
    w_l = W[:, :DH]
    w_r = W[:, DH:]
    return _matmul(h_l, h_r, w_l, w_r, b.reshape(1, D_OUT))
